# baseline (speedup 1.0000x reference)
import functools
import math

import jax
import jax.numpy as jnp
from jax.experimental import pallas as pl

_ACT_SPARSITY = 0.65
_INT32_MIN = jnp.iinfo(jnp.int32).min
_INT32_MAX = jnp.iinfo(jnp.int32).max


def _topk_mask_kernel(x_ref, o_ref, *, k):
    x = x_ref[...]
    u = jax.lax.bitcast_convert_type(x, jnp.int32)
    key = jnp.where(u < 0, u ^ jnp.int32(0x7FFFFFFF), u)

    rows, n = x.shape
    lo0 = jnp.full((rows, 1), _INT32_MIN, dtype=jnp.int32)
    hi0 = jnp.full((rows, 1), _INT32_MAX, dtype=jnp.int32)
    ones = jnp.ones((n, 1), dtype=jnp.float32)
    kf = jnp.float32(k)

    def body(_, carry):
        lo, hi = carry
        xor = lo ^ hi
        mid = (lo & hi) + (xor >> 1) + (xor & 1)
        mask = (key >= mid).astype(jnp.float32)
        cnt = jax.lax.dot_general(
            mask, ones, (((1,), (0,)), ((), ())),
            preferred_element_type=jnp.float32)
        ge = cnt >= kf
        eq = cnt == kf
        lo = jnp.where(ge, mid, lo)
        hi = jnp.where(eq, mid, jnp.where(ge, hi, mid - 1))
        return lo, hi

    lo, _ = jax.lax.fori_loop(0, 32, body, (lo0, hi0))
    o_ref[...] = jnp.where(key >= lo, x, 0.0)


def kernel(inputs):
    out_shape = inputs.shape
    x = inputs.reshape(inputs.shape[0], -1)
    m, n = x.shape
    k = math.floor((1.0 - _ACT_SPARSITY) * n)

    block = 256
    while m % block:
        block //= 2
    grid = m // block

    out = pl.pallas_call(
        functools.partial(_topk_mask_kernel, k=k),
        grid=(grid,),
        in_specs=[pl.BlockSpec((block, n), lambda i: (i, 0))],
        out_specs=pl.BlockSpec((block, n), lambda i: (i, 0)),
        out_shape=jax.ShapeDtypeStruct((m, n), x.dtype),
    )(x)
    return out.reshape(out_shape)


# speedup vs baseline: 1.5253x; 1.5253x over previous
import functools
import math

import jax
import jax.numpy as jnp
from jax.experimental import pallas as pl

_ACT_SPARSITY = 0.65
_INT32_MIN = jnp.iinfo(jnp.int32).min
_INT32_MAX = jnp.iinfo(jnp.int32).max


def _topk_mask_kernel(x_ref, o_ref, *, k):
    x = x_ref[...]
    u = jax.lax.bitcast_convert_type(x, jnp.int32)
    key = jnp.where(u < 0, u ^ jnp.int32(0x7FFFFFFF), u)

    rows, n = x.shape
    lo0 = jnp.full((rows, 1), _INT32_MIN, dtype=jnp.int32)
    hi0 = jnp.full((rows, 1), _INT32_MAX, dtype=jnp.int32)
    def cond(carry):
        i, lo, hi = carry
        return (i < 32) & jnp.any(lo < hi)

    def body(carry):
        i, lo, hi = carry
        xor = lo ^ hi
        mid = (lo & hi) + (xor >> 1) + (xor & 1)
        cnt = jnp.sum((key >= mid).astype(jnp.int32), axis=1, keepdims=True)
        ge = cnt >= k
        eq = cnt == k
        lo = jnp.where(ge, mid, lo)
        hi = jnp.where(eq, mid, jnp.where(ge, hi, mid - 1))
        return i + 1, lo, hi

    _, lo, _ = jax.lax.while_loop(cond, body, (jnp.int32(0), lo0, hi0))
    o_ref[...] = jnp.where(key >= lo, x, 0.0)


def kernel(inputs):
    out_shape = inputs.shape
    x = inputs.reshape(inputs.shape[0], -1)
    m, n = x.shape
    k = math.floor((1.0 - _ACT_SPARSITY) * n)

    block = 256
    while m % block:
        block //= 2
    grid = m // block

    out = pl.pallas_call(
        functools.partial(_topk_mask_kernel, k=k),
        grid=(grid,),
        in_specs=[pl.BlockSpec((block, n), lambda i: (i, 0))],
        out_specs=pl.BlockSpec((block, n), lambda i: (i, 0)),
        out_shape=jax.ShapeDtypeStruct((m, n), x.dtype),
    )(x)
    return out.reshape(out_shape)
